# baseline (speedup 1.0000x reference)
import functools
import jax
import jax.numpy as jnp
from jax.experimental import pallas as pl
from jax.experimental.pallas import tpu as pltpu

_RADIUS2 = 0.25 * 0.25
_K = 16
_BIG = 1e9
_NSEG = 16


def _sa_body(xyz_blk, xyzT_ref, feats_ref, W1_ref, b1_ref, W2_ref, b2_ref,
             out_blk, h_scr, key_scr, found_scr):
    @pl.when(pl.program_id(1) == 0)
    def _compute_mlp():
        f = feats_ref[0]
        h1 = jax.lax.dot_general(W1_ref[...], f, (((1,), (0,)), ((), ())),
                                 preferred_element_type=jnp.float32)
        h1 = jnp.maximum(h1 + b1_ref[...], 0.0)
        h2 = jax.lax.dot_general(W2_ref[...], h1, (((1,), (0,)), ((), ())),
                                 preferred_element_type=jnp.float32)
        h_scr[...] = jnp.maximum(h2 + b2_ref[...], 0.0).astype(jnp.bfloat16)

    cx = xyz_blk[0]
    xT = xyzT_ref[0]
    d0 = cx[:, 0:1] - xT[0:1, :]
    d1 = cx[:, 1:2] - xT[1:2, :]
    d2c = cx[:, 2:3] - xT[2:3, :]
    d2 = (d0 * d0 + d1 * d1) + d2c * d2c

    jidx = jax.lax.broadcasted_iota(jnp.int32, d2.shape, 1).astype(jnp.float32)
    key_scr[...] = jnp.where(d2 < _RADIUS2, jidx, _BIG)
    out_blk[0] = jnp.zeros_like(out_blk[0])
    mi = key_scr.shape[0]
    n = key_scr.shape[1]
    found_scr[...] = jnp.zeros((mi, 1), jnp.float32)

    nq = n // _NSEG
    for q in range(_NSEG):
        qs = q * nq

        def cond(alive):
            return alive

        def body(alive):
            key = key_scr[:, qs:qs + nq]
            jmin = jnp.min(key, axis=1, keepdims=True)
            got = jmin < (_BIG * 0.5)
            found = found_scr[...]
            filled = got & (found < float(_K))
            oh = (key == jmin).astype(jnp.bfloat16)
            g = jax.lax.dot_general(
                oh, h_scr[:, qs:qs + nq], (((1,), (1,)), ((), ())),
                preferred_element_type=jnp.float32)
            cur = out_blk[0]
            out_blk[0] = jnp.where(filled, jnp.maximum(cur, g), cur)
            key_scr[:, qs:qs + nq] = key + oh.astype(jnp.float32) * _BIG
            found_scr[...] = found + jnp.where(got, 1.0, 0.0)
            return jnp.min(jmin) < (_BIG * 0.5)

        jax.lax.while_loop(cond, body, True)


def kernel(xyz, feats, W1, b1, W2, b2):
    bs, n, _ = xyz.shape
    c_in = feats.shape[1]
    c = W1.shape[0]
    m = n
    mi = min(1024, m)
    xyzT = jnp.transpose(xyz, (0, 2, 1))
    b1c = b1[:, None]
    b2c = b2[:, None]

    grid = (bs, m // mi)
    out = pl.pallas_call(
        _sa_body,
        grid=grid,
        in_specs=[
            pl.BlockSpec((1, mi, 3), lambda b, mb: (b, mb, 0)),
            pl.BlockSpec((1, 3, n), lambda b, mb: (b, 0, 0)),
            pl.BlockSpec((1, c_in, n), lambda b, mb: (b, 0, 0)),
            pl.BlockSpec((c, c_in), lambda b, mb: (0, 0)),
            pl.BlockSpec((c, 1), lambda b, mb: (0, 0)),
            pl.BlockSpec((c, c), lambda b, mb: (0, 0)),
            pl.BlockSpec((c, 1), lambda b, mb: (0, 0)),
        ],
        out_specs=pl.BlockSpec((1, mi, c), lambda b, mb: (b, mb, 0)),
        out_shape=jax.ShapeDtypeStruct((bs, m, c), jnp.float32),
        scratch_shapes=[
            pltpu.VMEM((c, n), jnp.bfloat16),
            pltpu.VMEM((mi, n), jnp.float32),
            pltpu.VMEM((mi, 1), jnp.float32),
        ],
    )(xyz, xyzT, feats, W1, b1c, W2, b2c)

    child_feats = jnp.transpose(out, (0, 2, 1))
    return (xyz, child_feats)


# speedup vs baseline: 1.0095x; 1.0095x over previous
import functools
import jax
import jax.numpy as jnp
from jax.experimental import pallas as pl
from jax.experimental.pallas import tpu as pltpu

_RADIUS2 = 0.25 * 0.25
_K = 16
_BIG = 1e9
_NSEG = 8


def _sa_body(xyz_blk, xyzT_ref, feats_ref, W1_ref, b1_ref, W2_ref, b2_ref,
             out_blk, h_scr, key_scr, found_scr):
    @pl.when(pl.program_id(1) == 0)
    def _compute_mlp():
        f = feats_ref[0]
        h1 = jax.lax.dot_general(W1_ref[...], f, (((1,), (0,)), ((), ())),
                                 preferred_element_type=jnp.float32)
        h1 = jnp.maximum(h1 + b1_ref[...], 0.0)
        h2 = jax.lax.dot_general(W2_ref[...], h1, (((1,), (0,)), ((), ())),
                                 preferred_element_type=jnp.float32)
        h_scr[...] = jnp.maximum(h2 + b2_ref[...], 0.0).astype(jnp.bfloat16)

    cx = xyz_blk[0]
    xT = xyzT_ref[0]
    d0 = cx[:, 0:1] - xT[0:1, :]
    d1 = cx[:, 1:2] - xT[1:2, :]
    d2c = cx[:, 2:3] - xT[2:3, :]
    d2 = (d0 * d0 + d1 * d1) + d2c * d2c

    jidx = jax.lax.broadcasted_iota(jnp.int32, d2.shape, 1).astype(jnp.float32)
    key_scr[...] = jnp.where(d2 < _RADIUS2, jidx, _BIG)
    out_blk[0] = jnp.zeros_like(out_blk[0])
    mi = key_scr.shape[0]
    n = key_scr.shape[1]
    found_scr[...] = jnp.zeros((mi, 1), jnp.float32)

    nq = n // _NSEG
    for q in range(_NSEG):
        qs = q * nq

        def cond(alive):
            return alive

        def body(alive):
            key = key_scr[:, qs:qs + nq]
            jmin = jnp.min(key, axis=1, keepdims=True)
            got_any = jnp.min(jmin) < (_BIG * 0.5)

            @pl.when(got_any)
            def _extract():
                got = jmin < (_BIG * 0.5)
                found = found_scr[...]
                filled = got & (found < float(_K))
                oh = (key == jmin).astype(jnp.bfloat16)
                g = jax.lax.dot_general(
                    oh, h_scr[:, qs:qs + nq], (((1,), (1,)), ((), ())),
                    preferred_element_type=jnp.float32)
                cur = out_blk[0]
                out_blk[0] = jnp.where(filled, jnp.maximum(cur, g), cur)
                key_scr[:, qs:qs + nq] = key + oh.astype(jnp.float32) * _BIG
                found_scr[...] = found + jnp.where(got, 1.0, 0.0)

            return got_any

        jax.lax.while_loop(cond, body, True)


def kernel(xyz, feats, W1, b1, W2, b2):
    bs, n, _ = xyz.shape
    c_in = feats.shape[1]
    c = W1.shape[0]
    m = n
    mi = min(1024, m)
    xyzT = jnp.transpose(xyz, (0, 2, 1))
    b1c = b1[:, None]
    b2c = b2[:, None]

    grid = (bs, m // mi)
    out = pl.pallas_call(
        _sa_body,
        grid=grid,
        in_specs=[
            pl.BlockSpec((1, mi, 3), lambda b, mb: (b, mb, 0)),
            pl.BlockSpec((1, 3, n), lambda b, mb: (b, 0, 0)),
            pl.BlockSpec((1, c_in, n), lambda b, mb: (b, 0, 0)),
            pl.BlockSpec((c, c_in), lambda b, mb: (0, 0)),
            pl.BlockSpec((c, 1), lambda b, mb: (0, 0)),
            pl.BlockSpec((c, c), lambda b, mb: (0, 0)),
            pl.BlockSpec((c, 1), lambda b, mb: (0, 0)),
        ],
        out_specs=pl.BlockSpec((1, mi, c), lambda b, mb: (b, mb, 0)),
        out_shape=jax.ShapeDtypeStruct((bs, m, c), jnp.float32),
        scratch_shapes=[
            pltpu.VMEM((c, n), jnp.bfloat16),
            pltpu.VMEM((mi, n), jnp.float32),
            pltpu.VMEM((mi, 1), jnp.float32),
        ],
    )(xyz, xyzT, feats, W1, b1c, W2, b2c)

    child_feats = jnp.transpose(out, (0, 2, 1))
    return (xyz, child_feats)


# speedup vs baseline: 1.0690x; 1.0589x over previous
import functools
import jax
import jax.numpy as jnp
from jax.experimental import pallas as pl
from jax.experimental.pallas import tpu as pltpu

_RADIUS2 = 0.25 * 0.25
_K = 16
_BIG = 1e9
_NSEG = 8


def _sa_body(xyz_blk, xyzT_ref, feats_ref, W1_ref, b1_ref, W2_ref, b2_ref,
             out_blk, h_scr, key_scr, found_scr):
    @pl.when(pl.program_id(1) == 0)
    def _compute_mlp():
        f = feats_ref[0]
        h1 = jax.lax.dot_general(W1_ref[...], f, (((1,), (0,)), ((), ())),
                                 preferred_element_type=jnp.float32)
        h1 = jnp.maximum(h1 + b1_ref[...], 0.0)
        h2 = jax.lax.dot_general(W2_ref[...], h1, (((1,), (0,)), ((), ())),
                                 preferred_element_type=jnp.float32)
        h_scr[...] = jnp.maximum(h2 + b2_ref[...], 0.0).astype(jnp.bfloat16)

    cx = xyz_blk[0]
    xT = xyzT_ref[0]
    d0 = cx[:, 0:1] - xT[0:1, :]
    d1 = cx[:, 1:2] - xT[1:2, :]
    d2c = cx[:, 2:3] - xT[2:3, :]
    d2 = (d0 * d0 + d1 * d1) + d2c * d2c

    jidx = jax.lax.broadcasted_iota(jnp.int32, d2.shape, 1).astype(jnp.float32)
    key_scr[...] = jnp.where(d2 < _RADIUS2, jidx, _BIG)
    out_blk[0] = jnp.zeros_like(out_blk[0])
    mi = key_scr.shape[0]
    n = key_scr.shape[1]
    found_scr[...] = jnp.zeros((mi, 1), jnp.float32)

    nq = n // _NSEG
    for q in range(_NSEG):
        qs = q * nq

        def cond(alive):
            return alive

        def body(alive):
            key = key_scr[:, qs:qs + nq]
            jmin = jnp.min(key, axis=1, keepdims=True)
            got = jmin < (_BIG * 0.5)
            found = found_scr[...]
            filled = got & (found < float(_K))
            oh = (key == jmin).astype(jnp.bfloat16)
            g = jax.lax.dot_general(
                oh, h_scr[:, qs:qs + nq], (((1,), (1,)), ((), ())),
                preferred_element_type=jnp.float32)
            cur = out_blk[0]
            out_blk[0] = jnp.where(filled, jnp.maximum(cur, g), cur)
            key_scr[:, qs:qs + nq] = key + oh.astype(jnp.float32) * _BIG
            found_scr[...] = found + jnp.where(got, 1.0, 0.0)
            return jnp.min(jmin) < (_BIG * 0.5)

        jax.lax.while_loop(cond, body, True)


def kernel(xyz, feats, W1, b1, W2, b2):
    bs, n, _ = xyz.shape
    c_in = feats.shape[1]
    c = W1.shape[0]
    m = n
    mi = min(1024, m)
    xyzT = jnp.transpose(xyz, (0, 2, 1))
    b1c = b1[:, None]
    b2c = b2[:, None]

    grid = (bs, m // mi)
    out = pl.pallas_call(
        _sa_body,
        grid=grid,
        in_specs=[
            pl.BlockSpec((1, mi, 3), lambda b, mb: (b, mb, 0)),
            pl.BlockSpec((1, 3, n), lambda b, mb: (b, 0, 0)),
            pl.BlockSpec((1, c_in, n), lambda b, mb: (b, 0, 0)),
            pl.BlockSpec((c, c_in), lambda b, mb: (0, 0)),
            pl.BlockSpec((c, 1), lambda b, mb: (0, 0)),
            pl.BlockSpec((c, c), lambda b, mb: (0, 0)),
            pl.BlockSpec((c, 1), lambda b, mb: (0, 0)),
        ],
        out_specs=pl.BlockSpec((1, mi, c), lambda b, mb: (b, mb, 0)),
        out_shape=jax.ShapeDtypeStruct((bs, m, c), jnp.float32),
        scratch_shapes=[
            pltpu.VMEM((c, n), jnp.bfloat16),
            pltpu.VMEM((mi, n), jnp.float32),
            pltpu.VMEM((mi, 1), jnp.float32),
        ],
    )(xyz, xyzT, feats, W1, b1c, W2, b2c)

    child_feats = jnp.transpose(out, (0, 2, 1))
    return (xyz, child_feats)
